# Initial kernel scaffold; baseline (speedup 1.0000x reference)
import jax
import jax.numpy as jnp
from jax.experimental import pallas as pl


def kernel(indices, table):
    raise NotImplementedError("write your pallas kernel here")


# speedup vs baseline: 4.7370x; 4.7370x over previous
import functools

import jax
import jax.numpy as jnp
from jax import lax
from jax.experimental import pallas as pl
from jax.experimental.pallas import tpu as pltpu
from jax.experimental.pallas import tpu_sc as plsc

EMBEDDING_DIM = 64
TOTAL = 16384 * 200
NUM_CORES = 2
NUM_SUBCORES = 16
NUM_WORKERS = NUM_CORES * NUM_SUBCORES
PER_WORKER = TOTAL // NUM_WORKERS
CHUNK = 512
NUM_CHUNKS = PER_WORKER // CHUNK

_mesh = plsc.VectorSubcoreMesh(core_axis_name="c", subcore_axis_name="s")


@functools.partial(
    pl.kernel,
    mesh=_mesh,
    out_type=jax.ShapeDtypeStruct((TOTAL, EMBEDDING_DIM), jnp.float32),
    scratch_types=[
        pltpu.VMEM((CHUNK,), jnp.int32),
        pltpu.VMEM((CHUNK, EMBEDDING_DIM), jnp.float32),
        pltpu.SemaphoreType.DMA,
    ],
    compiler_params=pltpu.CompilerParams(use_tc_tiling_on_sc=False),
)
def _emb_lookup(idx_hbm, table_hbm, out_hbm, idx_v, rows_v, sem):
    wid = lax.axis_index("s") * NUM_CORES + lax.axis_index("c")
    base = wid * PER_WORKER

    def body(i, _):
        off = base + i * CHUNK
        pltpu.sync_copy(idx_hbm.at[pl.ds(off, CHUNK)], idx_v)
        pltpu.async_copy(table_hbm.at[idx_v], rows_v, sem).wait()
        pltpu.sync_copy(rows_v, out_hbm.at[pl.ds(off, CHUNK)])
        return 0

    lax.fori_loop(0, NUM_CHUNKS, body, 0)


def kernel(indices, table):
    flat = indices.reshape(TOTAL)
    out = _emb_lookup(flat, table)
    return out.reshape(indices.shape[0], indices.shape[1], EMBEDDING_DIM)


# speedup vs baseline: 5.1699x; 1.0914x over previous
import functools

import jax
import jax.numpy as jnp
from jax import lax
from jax.experimental import pallas as pl
from jax.experimental.pallas import tpu as pltpu
from jax.experimental.pallas import tpu_sc as plsc

EMBEDDING_DIM = 64
TOTAL = 16384 * 200
NUM_CORES = 2
NUM_SUBCORES = 16
NUM_WORKERS = NUM_CORES * NUM_SUBCORES
CHUNK = 512
CHUNKS_TOTAL = TOTAL // CHUNK
CHUNKS_PER_W = CHUNKS_TOTAL // NUM_WORKERS
CPS = 20
NUM_SUPERS = CHUNKS_PER_W // CPS

_mesh = plsc.VectorSubcoreMesh(core_axis_name="c", subcore_axis_name="s")


@functools.partial(
    pl.kernel,
    mesh=_mesh,
    out_type=jax.ShapeDtypeStruct((TOTAL, EMBEDDING_DIM), jnp.float32),
    scratch_types=[
        pltpu.VMEM((CPS * CHUNK,), jnp.int32),
        pltpu.VMEM((CPS * CHUNK,), jnp.int32),
        pltpu.VMEM((CHUNK, EMBEDDING_DIM), jnp.float32),
        pltpu.VMEM((CHUNK, EMBEDDING_DIM), jnp.float32),
        pltpu.SemaphoreType.DMA,
        pltpu.SemaphoreType.DMA,
        pltpu.SemaphoreType.DMA,
        pltpu.SemaphoreType.DMA,
    ],
    compiler_params=pltpu.CompilerParams(use_tc_tiling_on_sc=False),
)
def _emb_lookup(idx_hbm, table_hbm, out_hbm,
                idx_v0, idx_v1, rows0, rows1, sg0, sg1, ss0, ss1):
    wid = lax.axis_index("s") * NUM_CORES + lax.axis_index("c")
    base = wid * CHUNKS_PER_W * CHUNK
    idxb = (idx_v0, idx_v1)
    rows = (rows0, rows1)
    sg = (sg0, sg1)
    ss = (ss0, ss1)
    SUPER_N = CPS * CHUNK

    def idx_slot(sb, slot):
        off = pl.multiple_of(slot * CHUNK, CHUNK)
        return idxb[sb].at[pl.ds(off, CHUNK)]

    def gather_wait(b):
        pltpu.make_async_copy(table_hbm.at[idx_slot(0, 0)], rows[b], sg[b]).wait()

    def store_wait(b):
        pltpu.make_async_copy(rows[b], out_hbm.at[pl.ds(base, CHUNK)], ss[b]).wait()

    pltpu.sync_copy(idx_hbm.at[pl.ds(base, SUPER_N)], idx_v0)
    pltpu.async_copy(table_hbm.at[idx_slot(0, 0)], rows1, sg1)
    pltpu.async_copy(rows0, out_hbm.at[pl.ds(base, CHUNK)], ss0)

    def super_pair(sp, _):
        for sb in (0, 1):
            s = 2 * sp + sb
            soff = pl.multiple_of(base + s * SUPER_N, CHUNK)
            pltpu.sync_copy(idx_hbm.at[pl.ds(soff, SUPER_N)], idxb[sb])

            def chunk_pair(p, _):
                for b in (0, 1):
                    slot = 2 * p + b
                    c = s * CPS + slot
                    store_wait(b)
                    pltpu.async_copy(table_hbm.at[idx_slot(sb, slot)], rows[b], sg[b])
                    gather_wait(1 - b)
                    prev = pl.multiple_of(
                        base + jnp.maximum(c - 1, 0) * CHUNK, CHUNK)
                    pltpu.async_copy(rows[1 - b], out_hbm.at[pl.ds(prev, CHUNK)], ss[1 - b])
                return 0

            lax.fori_loop(0, CPS // 2, chunk_pair, 0)
        return 0

    lax.fori_loop(0, NUM_SUPERS // 2, super_pair, 0)

    gather_wait(1)
    pltpu.async_copy(
        rows1, out_hbm.at[pl.ds(base + (CHUNKS_PER_W - 1) * CHUNK, CHUNK)], ss1)
    store_wait(0)
    store_wait(1)


def kernel(indices, table):
    flat = indices.reshape(TOTAL)
    out = _emb_lookup(flat, table)
    return out.reshape(indices.shape[0], indices.shape[1], EMBEDDING_DIM)
